# baseline (speedup 1.0000x reference)
import functools

import jax
import jax.numpy as jnp
from jax import lax
from jax.experimental import pallas as pl
from jax.experimental.pallas import tpu as pltpu
from jax.experimental.pallas import tpu_sc as plsc

_VOCAB_BLOCK = 2048


def _gather_sc(table128, idx, D):
    B = idx.shape[0]
    NW = 32
    bpw = B // NW
    L = 16

    mesh = plsc.VectorSubcoreMesh(core_axis_name="c", subcore_axis_name="s")

    @functools.partial(
        pl.kernel,
        mesh=mesh,
        out_type=jax.ShapeDtypeStruct((B * D,), jnp.float32),
        scratch_types=[
            pltpu.VMEM((bpw,), jnp.int32),
            pltpu.VMEM((bpw,), jnp.int32),
            pltpu.VMEM((bpw, 128), jnp.float32),
            pltpu.VMEM((bpw * D,), jnp.float32),
            pltpu.SemaphoreType.DMA,
        ],
    )
    def gk(table_hbm, idx_hbm, out_hbm, idx_v, q_v, rows_v, out_v, sem):
        wid = lax.axis_index("s") * 2 + lax.axis_index("c")
        base = wid * bpw
        pltpu.sync_copy(idx_hbm.at[pl.ds(base, bpw)], idx_v)
        for g in range(bpw // L):
            v = idx_v[pl.ds(g * L, L)]
            q_v[pl.ds(g * L, L)] = lax.shift_right_logical(v, 3)
        pltpu.async_copy(table_hbm.at[q_v], rows_v, sem).wait()
        for g in range(bpw // L):
            offs = (idx_v[pl.ds(g * L, L)] & 7) * D
            for r in range(L):
                j = g * L + r
                out_v[pl.ds(j * D, D)] = rows_v[j, pl.ds(offs[r], D)]
        pltpu.sync_copy(out_v, out_hbm.at[pl.ds(base * D, bpw * D)])

    return gk(table128, idx).reshape(B, D)


def _matmul_body(x_ref, w_ref, o_ref):
    o_ref[...] = jnp.dot(x_ref[...], w_ref[...],
                         preferred_element_type=jnp.float32)


def _project_tc(xc, wc):
    B, K = xc.shape
    V = wc.shape[1]
    grid = pl.cdiv(V, _VOCAB_BLOCK)
    return pl.pallas_call(
        _matmul_body,
        grid=(grid,),
        in_specs=[
            pl.BlockSpec((B, K), lambda j: (0, 0)),
            pl.BlockSpec((K, _VOCAB_BLOCK), lambda j: (0, j)),
        ],
        out_specs=pl.BlockSpec((B, _VOCAB_BLOCK), lambda j: (0, j)),
        out_shape=jax.ShapeDtypeStruct((B, V), jnp.float32),
        compiler_params=pltpu.CompilerParams(
            dimension_semantics=("arbitrary",)),
    )(xc, wc)


def _split_hi_lo(a):
    hi = a.astype(jnp.bfloat16)
    lo = (a - hi.astype(jnp.float32)).astype(jnp.bfloat16)
    return hi, lo


def kernel(center_word, emb_table, W, b):
    V, D = emb_table.shape
    table128 = emb_table.reshape(V * D // 128, 128)
    x = _gather_sc(table128, center_word, D)
    x_hi, x_lo = _split_hi_lo(x)
    ones = jnp.ones((x.shape[0], 1), jnp.bfloat16)
    xc = jnp.concatenate([x_hi, x_hi, x_lo, ones, ones], axis=1)

    Wt = W.T
    w_hi, w_lo = _split_hi_lo(Wt)
    b_hi, b_lo = _split_hi_lo(b[None, :])
    wc = jnp.concatenate([w_hi, w_lo, w_hi, b_hi, b_lo], axis=0)

    return _project_tc(xc, wc)


# speedup vs baseline: 1.0102x; 1.0102x over previous
import functools

import jax
import jax.numpy as jnp
from jax import lax
from jax.experimental import pallas as pl
from jax.experimental.pallas import tpu as pltpu
from jax.experimental.pallas import tpu_sc as plsc

_VOCAB_BLOCK = 2048


def _gather_sc(table128, idx, D):
    B = idx.shape[0]
    NW = 32
    bpw = B // NW
    L = 16

    mesh = plsc.VectorSubcoreMesh(core_axis_name="c", subcore_axis_name="s")

    @functools.partial(
        pl.kernel,
        mesh=mesh,
        out_type=jax.ShapeDtypeStruct((B, D), jnp.float32),
        scratch_types=[
            pltpu.VMEM((bpw,), jnp.int32),
            pltpu.VMEM((bpw,), jnp.int32),
            pltpu.VMEM((bpw, 128), jnp.float32),
            pltpu.VMEM((bpw, D), jnp.float32),
            pltpu.SemaphoreType.DMA,
        ],
    )
    def gk(table_hbm, idx_hbm, out_hbm, idx_v, q_v, rows_v, out_v, sem):
        wid = lax.axis_index("s") * 2 + lax.axis_index("c")
        base = wid * bpw
        pltpu.sync_copy(idx_hbm.at[pl.ds(base, bpw)], idx_v)
        for g in range(bpw // L):
            v = idx_v[pl.ds(g * L, L)]
            q_v[pl.ds(g * L, L)] = lax.shift_right_logical(v, 3)
        pltpu.async_copy(table_hbm.at[q_v], rows_v, sem).wait()
        for g in range(bpw // L):
            offs = (idx_v[pl.ds(g * L, L)] & 7) * D
            for r in range(L):
                j = g * L + r
                out_v[j, :] = rows_v[j, pl.ds(offs[r], D)]
        pltpu.sync_copy(out_v, out_hbm.at[pl.ds(base, bpw)])

    return gk(table128, idx)


def _matmul_body(x_ref, w_ref, b_ref, o_ref, xc_ref):
    @pl.when(pl.program_id(0) == 0)
    def _():
        x = x_ref[...]
        x_hi = x.astype(jnp.bfloat16)
        x_lo = (x - x_hi.astype(jnp.float32)).astype(jnp.bfloat16)
        xc_ref[:, 0:16] = x_hi
        xc_ref[:, 16:32] = x_hi
        xc_ref[:, 32:48] = x_lo

    w = w_ref[...]
    w_hi = w.astype(jnp.bfloat16)
    w_lo = (w - w_hi.astype(jnp.float32)).astype(jnp.bfloat16)
    wc = jnp.concatenate([w_hi, w_lo, w_hi], axis=1)
    acc = lax.dot_general(xc_ref[...], wc, (((1,), (1,)), ((), ())),
                          preferred_element_type=jnp.float32)
    o_ref[...] = acc + b_ref[...]


def _project_tc(x, W, b2):
    B = x.shape[0]
    V, D = W.shape
    grid = pl.cdiv(V, _VOCAB_BLOCK)
    return pl.pallas_call(
        _matmul_body,
        grid=(grid,),
        in_specs=[
            pl.BlockSpec((B, D), lambda j: (0, 0)),
            pl.BlockSpec((_VOCAB_BLOCK, D), lambda j: (j, 0)),
            pl.BlockSpec((1, _VOCAB_BLOCK), lambda j: (0, j)),
        ],
        out_specs=pl.BlockSpec((B, _VOCAB_BLOCK), lambda j: (0, j)),
        out_shape=jax.ShapeDtypeStruct((B, V), jnp.float32),
        scratch_shapes=[pltpu.VMEM((B, 3 * D), jnp.bfloat16)],
        compiler_params=pltpu.CompilerParams(
            dimension_semantics=("arbitrary",)),
    )(x, W, b2)


def kernel(center_word, emb_table, W, b):
    V, D = emb_table.shape
    table128 = emb_table.reshape(V * D // 128, 128)
    x = _gather_sc(table128, center_word, D)
    return _project_tc(x, W, b[None, :])


# speedup vs baseline: 2.5337x; 2.5082x over previous
import functools

import jax
import jax.numpy as jnp
from jax import lax
from jax.experimental import pallas as pl
from jax.experimental.pallas import tpu as pltpu
from jax.experimental.pallas import tpu_sc as plsc

_VOCAB_BLOCK = 2048


def _gather_sc(table128, idx, D):
    B = idx.shape[0]
    NW = 32
    bpw = B // NW
    L = 16

    mesh = plsc.VectorSubcoreMesh(core_axis_name="c", subcore_axis_name="s")

    @functools.partial(
        pl.kernel,
        mesh=mesh,
        out_type=jax.ShapeDtypeStruct((B, D), jnp.float32),
        scratch_types=[
            pltpu.VMEM((bpw,), jnp.int32),
            pltpu.VMEM((bpw,), jnp.int32),
            pltpu.VMEM((bpw, 128), jnp.float32),
            pltpu.VMEM((bpw, D), jnp.float32),
            pltpu.SemaphoreType.DMA,
        ],
    )
    def gk(table_hbm, idx_hbm, out_hbm, idx_v, q_v, rows_v, out_v, sem):
        wid = lax.axis_index("s") * 2 + lax.axis_index("c")
        base = wid * bpw
        pltpu.sync_copy(idx_hbm.at[pl.ds(base, bpw)], idx_v)
        for g in range(bpw // L):
            v = idx_v[pl.ds(g * L, L)]
            q_v[pl.ds(g * L, L)] = lax.shift_right_logical(v, 3)
        pltpu.async_copy(table_hbm.at[q_v], rows_v, sem).wait()
        for g in range(bpw // L):
            offs = (idx_v[pl.ds(g * L, L)] & 7) * D
            for r in range(L):
                j = g * L + r
                out_v[j, :] = rows_v[j, pl.ds(offs[r], D)]
        pltpu.sync_copy(out_v, out_hbm.at[pl.ds(base, bpw)])

    return gk(table128, idx)


def _matmul_body(x_ref, w_ref, b_ref, o_ref, xct_ref):
    @pl.when(pl.program_id(0) == 0)
    def _():
        xt = x_ref[...].T
        xt_hi = xt.astype(jnp.bfloat16)
        xt_lo = (xt - xt_hi.astype(jnp.float32)).astype(jnp.bfloat16)
        xct_ref[0:16, :] = xt_hi
        xct_ref[16:32, :] = xt_hi
        xct_ref[32:48, :] = xt_lo
        xct_ref[48:50, :] = jnp.ones((2, xt.shape[1]), jnp.bfloat16)

    w = w_ref[...]
    w_hi = w.astype(jnp.bfloat16)
    w_lo = (w - w_hi.astype(jnp.float32)).astype(jnp.bfloat16)
    b_row = b_ref[...]
    b_hi = b_row.astype(jnp.bfloat16)
    b_lo = (b_row - b_hi.astype(jnp.float32)).astype(jnp.bfloat16)
    wc = jnp.concatenate([w_hi, w_lo, w_hi, b_hi.T, b_lo.T], axis=1)
    o_ref[...] = lax.dot_general(wc, xct_ref[...], (((1,), (0,)), ((), ())),
                                 preferred_element_type=jnp.float32)


def _project_tc(x, W, b2):
    B = x.shape[0]
    V, D = W.shape
    grid = pl.cdiv(V, _VOCAB_BLOCK)
    out_t = pl.pallas_call(
        _matmul_body,
        grid=(grid,),
        in_specs=[
            pl.BlockSpec((B, D), lambda j: (0, 0)),
            pl.BlockSpec((_VOCAB_BLOCK, D), lambda j: (j, 0)),
            pl.BlockSpec((1, _VOCAB_BLOCK), lambda j: (0, j)),
        ],
        out_specs=pl.BlockSpec((_VOCAB_BLOCK, B), lambda j: (j, 0)),
        out_shape=jax.ShapeDtypeStruct((V, B), jnp.float32),
        scratch_shapes=[pltpu.VMEM((3 * D + 2, B), jnp.bfloat16)],
        compiler_params=pltpu.CompilerParams(
            dimension_semantics=("arbitrary",)),
    )(x, W, b2)
    return out_t.T


def kernel(center_word, emb_table, W, b):
    V, D = emb_table.shape
    table128 = emb_table.reshape(V * D // 128, 128)
    x = _gather_sc(table128, center_word, D)
    return _project_tc(x, W, b[None, :])


# speedup vs baseline: 2.7576x; 1.0884x over previous
import functools

import jax
import jax.numpy as jnp
from jax import lax
from jax.experimental import pallas as pl
from jax.experimental.pallas import tpu as pltpu
from jax.experimental.pallas import tpu_sc as plsc

_VOCAB_BLOCK = 2048


def _gather_sc(table, idx):
    B = idx.shape[0]
    D = table.shape[1]
    NW = 32
    bpw = B // NW
    L = 16

    mesh = plsc.VectorSubcoreMesh(core_axis_name="c", subcore_axis_name="s")

    @functools.partial(
        pl.kernel,
        mesh=mesh,
        out_type=jax.ShapeDtypeStruct((B, D), jnp.float32),
        scratch_types=[
            pltpu.VMEM((bpw,), jnp.int32),
            pltpu.VMEM((bpw, D), jnp.float32),
            pltpu.SemaphoreType.DMA,
        ],
    )
    def gk(table_hbm, idx_hbm, out_hbm, idx_v, out_v, sem):
        wid = lax.axis_index("s") * 2 + lax.axis_index("c")
        base = wid * bpw
        pltpu.sync_copy(idx_hbm.at[pl.ds(base, bpw)], idx_v)
        cps = []
        for g in range(bpw // L):
            v = idx_v[pl.ds(g * L, L)]
            for r in range(L):
                j = g * L + r
                cps.append(pltpu.async_copy(
                    table_hbm.at[pl.ds(v[r], 1), :],
                    out_v.at[pl.ds(j, 1), :], sem))
        for cp in cps:
            cp.wait()
        pltpu.sync_copy(out_v, out_hbm.at[pl.ds(base, bpw)])

    return gk(table, idx)


def _matmul_body(x_ref, w_ref, b_ref, o_ref, xct_ref):
    @pl.when(pl.program_id(0) == 0)
    def _():
        xt = x_ref[...].T
        xt_hi = xt.astype(jnp.bfloat16)
        xt_lo = (xt - xt_hi.astype(jnp.float32)).astype(jnp.bfloat16)
        xct_ref[0:16, :] = xt_hi
        xct_ref[16:32, :] = xt_hi
        xct_ref[32:48, :] = xt_lo
        xct_ref[48:50, :] = jnp.ones((2, xt.shape[1]), jnp.bfloat16)

    w = w_ref[...]
    w_hi = w.astype(jnp.bfloat16)
    w_lo = (w - w_hi.astype(jnp.float32)).astype(jnp.bfloat16)
    b_row = b_ref[...]
    b_hi = b_row.astype(jnp.bfloat16)
    b_lo = (b_row - b_hi.astype(jnp.float32)).astype(jnp.bfloat16)
    wc = jnp.concatenate([w_hi, w_lo, w_hi, b_hi.T, b_lo.T], axis=1)
    o_ref[...] = lax.dot_general(wc, xct_ref[...], (((1,), (0,)), ((), ())),
                                 preferred_element_type=jnp.float32)


def _project_tc(x, W, b2):
    B = x.shape[0]
    V, D = W.shape
    grid = pl.cdiv(V, _VOCAB_BLOCK)
    out_t = pl.pallas_call(
        _matmul_body,
        grid=(grid,),
        in_specs=[
            pl.BlockSpec((B, D), lambda j: (0, 0)),
            pl.BlockSpec((_VOCAB_BLOCK, D), lambda j: (j, 0)),
            pl.BlockSpec((1, _VOCAB_BLOCK), lambda j: (0, j)),
        ],
        out_specs=pl.BlockSpec((_VOCAB_BLOCK, B), lambda j: (j, 0)),
        out_shape=jax.ShapeDtypeStruct((V, B), jnp.float32),
        scratch_shapes=[pltpu.VMEM((3 * D + 2, B), jnp.bfloat16)],
        compiler_params=pltpu.CompilerParams(
            dimension_semantics=("arbitrary",)),
    )(x, W, b2)
    return out_t.T


def kernel(center_word, emb_table, W, b):
    x = _gather_sc(emb_table, center_word)
    return _project_tc(x, W, b[None, :])


# speedup vs baseline: 3.8388x; 1.3921x over previous
import functools

import jax
import jax.numpy as jnp
from jax import lax
from jax.experimental import pallas as pl
from jax.experimental.pallas import tpu as pltpu
from jax.experimental.pallas import tpu_sc as plsc

_VOCAB_BLOCK = 2048


_TAIL = 32


def _gather_sc(embT, tail_flat, idx):
    D, V = embT.shape
    B = idx.shape[0]
    NW = 32
    bpw = B // NW
    L = 16
    TMAX = (V - 128) // 128
    TBASE = V - _TAIL

    mesh = plsc.VectorSubcoreMesh(core_axis_name="c", subcore_axis_name="s")

    @functools.partial(
        pl.kernel,
        mesh=mesh,
        out_type=jax.ShapeDtypeStruct((B, D), jnp.float32),
        scratch_types=[
            pltpu.VMEM((bpw,), jnp.int32),
            pltpu.VMEM((bpw,), jnp.int32),
            pltpu.VMEM((bpw * D + 1, 128), jnp.float32),
            pltpu.VMEM((_TAIL * D,), jnp.float32),
            pltpu.VMEM((bpw, D), jnp.float32),
            pltpu.SemaphoreType.DMA,
        ],
    )
    def gk(embT_hbm, tail_hbm, idx_hbm, out_hbm,
           idx_v, s_v, win_v, tail_v, out_v, sem):
        wid = lax.axis_index("s") * 2 + lax.axis_index("c")
        base = wid * bpw
        pltpu.sync_copy(idx_hbm.at[pl.ds(base, bpw)], idx_v)
        pltpu.sync_copy(tail_hbm, tail_v)
        for g in range(bpw // L):
            v = idx_v[pl.ds(g * L, L)]
            t = jnp.minimum(lax.shift_right_logical(v, 7), TMAX)
            s_v[pl.ds(g * L, L)] = t * 128
        cps = []
        for g in range(bpw // L):
            sv = s_v[pl.ds(g * L, L)]
            for r in range(L):
                j = g * L + r
                start = pl.multiple_of(sv[r], 128)
                cps.append(pltpu.async_copy(
                    embT_hbm.at[:, pl.ds(start, 128)],
                    win_v.at[pl.ds(j * D, D), :], sem))
        for cp in cps:
            cp.wait()
        lanes = lax.iota(jnp.int32, L)
        for g in range(bpw // L):
            v = idx_v[pl.ds(g * L, L)]
            cols = v - s_v[pl.ds(g * L, L)]
            for r in range(L):
                j = g * L + r
                c = cols[r]
                acc = jnp.zeros((L,), jnp.float32)
                for f in range(D):
                    v16 = win_v[j * D + f, pl.ds(c, L)]
                    acc = jnp.where(lanes == f, v16[0], acc)
                vr = v[r]
                toff = jnp.maximum(vr - TBASE, 0) * D
                tvec = tail_v[pl.ds(toff, L)]
                out_v[j, :] = jnp.where(vr >= TBASE, tvec, acc)
        pltpu.sync_copy(out_v, out_hbm.at[pl.ds(base, bpw)])

    return gk(embT, tail_flat, idx)


def _matmul_body(x_ref, wt_ref, b_ref, o_ref, xct_ref):
    @pl.when(pl.program_id(0) == 0)
    def _():
        xt = x_ref[...].T
        xt_hi = xt.astype(jnp.bfloat16)
        xt_lo = (xt - xt_hi.astype(jnp.float32)).astype(jnp.bfloat16)
        xct_ref[0:16, :] = xt_hi
        xct_ref[16:32, :] = xt_hi
        xct_ref[32:48, :] = xt_lo
        xct_ref[48:50, :] = jnp.ones((2, xt.shape[1]), jnp.bfloat16)

    wt = wt_ref[...]
    wt_hi = wt.astype(jnp.bfloat16)
    wt_lo = (wt - wt_hi.astype(jnp.float32)).astype(jnp.bfloat16)
    b_row = b_ref[...]
    b_hi = b_row.astype(jnp.bfloat16)
    b_lo = (b_row - b_hi.astype(jnp.float32)).astype(jnp.bfloat16)
    wct = jnp.concatenate([wt_hi, wt_lo, wt_hi, b_hi, b_lo], axis=0)
    o_ref[...] = lax.dot_general(wct, xct_ref[...], (((0,), (0,)), ((), ())),
                                 preferred_element_type=jnp.float32)


def _project_tc(x, wt, b2):
    B, D = x.shape
    V = wt.shape[1]
    grid = pl.cdiv(V, _VOCAB_BLOCK)
    out_t = pl.pallas_call(
        _matmul_body,
        grid=(grid,),
        in_specs=[
            pl.BlockSpec((B, D), lambda j: (0, 0)),
            pl.BlockSpec((D, _VOCAB_BLOCK), lambda j: (0, j)),
            pl.BlockSpec((1, _VOCAB_BLOCK), lambda j: (0, j)),
        ],
        out_specs=pl.BlockSpec((_VOCAB_BLOCK, B), lambda j: (j, 0)),
        out_shape=jax.ShapeDtypeStruct((V, B), jnp.float32),
        scratch_shapes=[pltpu.VMEM((3 * D + 2, B), jnp.bfloat16)],
        compiler_params=pltpu.CompilerParams(
            dimension_semantics=("arbitrary",)),
    )(x, wt, b2)
    return out_t.T


def kernel(center_word, emb_table, W, b):
    tail_flat = emb_table[emb_table.shape[0] - _TAIL:, :].reshape(-1)
    x = _gather_sc(emb_table.T, tail_flat, center_word)
    return _project_tc(x, W.T, b[None, :])
